# baseline (speedup 1.0000x reference)
import functools

import jax
import jax.numpy as jnp
from jax import lax
from jax.experimental import pallas as pl
from jax.experimental.pallas import tpu as pltpu
from jax.experimental.pallas import tpu_sc as plsc

NC, NS = 2, 16
NW = NC * NS
NBUF = 8
TBUF = 2
VBUF = 3
WBUF = 2


def _make_relayout(v, d):
    nb = 128
    full = v // nb
    rem = v - full * nb
    base, extra = full // NW, full % NW
    mesh = plsc.VectorSubcoreMesh(core_axis_name="c", subcore_axis_name="s")

    @functools.partial(
        pl.kernel,
        out_type=jax.ShapeDtypeStruct((v // 2, 2 * d), jnp.float32),
        mesh=mesh,
        scratch_types=[
            pltpu.VMEM((VBUF, d, nb), jnp.float32),
            pltpu.VMEM((WBUF, nb // 2, 2 * d + 1), jnp.float32),
            pltpu.SemaphoreType.DMA((VBUF,)),
            pltpu.SemaphoreType.DMA((WBUF,)),
        ],
        compiler_params=pltpu.CompilerParams(use_tc_tiling_on_sc=True,
                                             needs_layout_passes=False),
    )
    def relayout_kernel(tT_hbm, tail_hbm, out_hbm, src_v, dst_v, gsem, osem):
        wid = lax.axis_index("s") * NC + lax.axis_index("c")
        n = base + jnp.where(wid < extra, 1, 0)

        io16 = lax.iota(jnp.int32, 16)
        m_ids = [(io16 + 16 * g) >> 1 for g in range(8)]
        q_off = [((io16 + 16 * g) & 1) * d for g in range(8)]

        def blk_of(j):
            return j * NW + wid

        for p in range(VBUF):
            pltpu.async_copy(tT_hbm.at[:, pl.ds(blk_of(p) * nb, nb)],
                             src_v.at[p], gsem.at[p])

        def transpose_block(src, dst, ngroups):
            @pl.loop(0, d, step=2)
            def _row(dd0):
                for du in range(2):
                    dd = dd0 + du
                    for g in range(ngroups):
                        vals = src[dd, pl.ds(16 * g, 16)]
                        plsc.store_scatter(dst, [m_ids[g], q_off[g] + dd],
                                           vals)

        def write_out(w, m0, nrows):
            pltpu.async_copy(dst_v.at[w, pl.ds(0, nrows), pl.ds(0, 2 * d)],
                             out_hbm.at[pl.ds(m0, nrows)],
                             osem.at[w])

        def wait_out(w, m0, nrows):
            pltpu.make_async_copy(
                dst_v.at[w, pl.ds(0, nrows), pl.ds(0, 2 * d)],
                out_hbm.at[pl.ds(m0, nrows)],
                osem.at[w]).wait()

        @pl.loop(0, base + 1)
        def step(j):
            @pl.when(j < n)
            def _do():
                p = lax.rem(j, VBUF)
                w = lax.rem(j, WBUF)
                pltpu.make_async_copy(
                    tT_hbm.at[:, pl.ds(blk_of(j) * nb, nb)],
                    src_v.at[p], gsem.at[p]).wait()

                @pl.when(j >= WBUF)
                def _w():
                    wait_out(w, blk_of(j - WBUF) * (nb // 2), nb // 2)

                transpose_block(src_v.at[p], dst_v.at[w], 8)

                @pl.when(j + VBUF < n)
                def _refire():
                    pltpu.async_copy(
                        tT_hbm.at[:, pl.ds(blk_of(j + VBUF) * nb, nb)],
                        src_v.at[p], gsem.at[p])

                write_out(w, blk_of(j) * (nb // 2), nb // 2)

        @pl.loop(0, WBUF)
        def _drain(k):
            j = n - WBUF + k

            @pl.when(j >= 0)
            def _d():
                wait_out(lax.rem(j, WBUF), blk_of(j) * (nb // 2), nb // 2)

        if rem:
            @pl.when(wid == NW - 1)
            def _tail():
                pltpu.sync_copy(tail_hbm, src_v.at[0])
                transpose_block(src_v.at[0], dst_v.at[0], rem // 16)
                pltpu.sync_copy(
                    dst_v.at[0, pl.ds(0, rem // 2), pl.ds(0, 2 * d)],
                    out_hbm.at[pl.ds(full * (nb // 2), rem // 2)])

    return relayout_kernel


def _make_gather(bsz, seq, d):
    tb = bsz // 128
    assert tb == NW and d % 8 == 0 and seq % NBUF == 0
    d8 = d // 8
    mesh = plsc.VectorSubcoreMesh(core_axis_name="c", subcore_axis_name="s")

    @functools.partial(
        pl.kernel,
        out_type=jax.ShapeDtypeStruct((seq, d8, tb, 8, 128), jnp.float32),
        mesh=mesh,
        scratch_types=[
            pltpu.VMEM((seq // 8, 8, 128), jnp.int32),
            pltpu.VMEM((NBUF, 128, d), jnp.float32),
            pltpu.VMEM((TBUF, d8, 8, 129), jnp.float32),
            pltpu.SemaphoreType.DMA((NBUF,)),
            pltpu.SemaphoreType.DMA((TBUF,)),
        ],
        compiler_params=pltpu.CompilerParams(use_tc_tiling_on_sc=False,
                                             needs_layout_passes=False),
    )
    def gather_kernel(idx_hbm, table_hbm, out_hbm, idx_v, rows_v, t_v,
                      gsem, osem):
        wid = lax.axis_index("s") * NC + lax.axis_index("c")
        pltpu.sync_copy(idx_hbm.at[:, wid], idx_v)

        io16 = lax.iota(jnp.int32, 16)
        i0s = [(io16 + 16 * g) >> 3 for g in range(4)]
        i1s = [(io16 + 16 * g) & 7 for g in range(4)]

        for b in range(NBUF):
            pltpu.async_copy(table_hbm.at[idx_v.at[b // 8, b % 8]],
                             rows_v.at[b], gsem.at[b])

        @pl.loop(0, seq, step=NBUF)
        def step(j):
            for b in range(NBUF):
                i = j + b
                t = b % TBUF
                pltpu.make_async_copy(table_hbm.at[idx_v.at[i // 8, i % 8]],
                                      rows_v.at[b], gsem.at[b]).wait()

                @pl.when(i >= TBUF)
                def _wait_write():
                    pltpu.make_async_copy(
                        t_v.at[t, :, :, pl.ds(0, 128)],
                        out_hbm.at[i - TBUF, pl.ds(0, d8), wid],
                        osem.at[t]).wait()

                rows = rows_v.at[b]
                tdst = t_v.at[t]

                @pl.loop(0, 128, step=4)
                def _col(c0):
                    for cc in range(4):
                        c = c0 + cc
                        cvec = jnp.full((16,), c, jnp.int32)
                        for g in range(4):
                            vals = rows[c, pl.ds(16 * g, 16)]
                            plsc.store_scatter(
                                tdst, [i0s[g], i1s[g], cvec], vals)

                @pl.when(i + NBUF < seq)
                def _refire():
                    pltpu.async_copy(
                        table_hbm.at[idx_v.at[(i + NBUF) // 8,
                                              (i + NBUF) % 8]],
                        rows_v.at[b], gsem.at[b])

                pltpu.async_copy(t_v.at[t, :, :, pl.ds(0, 128)],
                                 out_hbm.at[i, pl.ds(0, d8), wid],
                                 osem.at[t])

        for k in range(TBUF):
            i = seq - TBUF + k
            pltpu.make_async_copy(t_v.at[i % TBUF, :, :, pl.ds(0, 128)],
                                  out_hbm.at[i, pl.ds(0, d8), wid],
                                  osem.at[i % TBUF]).wait()

    return gather_kernel


def kernel(inputs, table):
    bsz, seq = inputs.shape
    v, d = table.shape
    rem = v % 128
    tail = (jnp.pad(table[v - rem:].T, ((0, 0), (0, 128 - rem)))
            if rem else jnp.zeros((d, 128), table.dtype))
    tbl_lin = _make_relayout(v, d)(table.T, tail).reshape(v, d)
    idx4 = (inputs.astype(jnp.int32)
            .reshape(bsz // 128, 128, seq // 8, 8)
            .transpose(2, 0, 3, 1))
    out5 = _make_gather(bsz, seq, d)(idx4, tbl_lin)
    out = out5.transpose(2, 4, 0, 1, 3).reshape(bsz, seq, d)
    return out


# speedup vs baseline: 1.3737x; 1.3737x over previous
import functools

import jax
import jax.numpy as jnp
from jax import lax
from jax.experimental import pallas as pl
from jax.experimental.pallas import tpu as pltpu
from jax.experimental.pallas import tpu_sc as plsc

NC, NS = 2, 16
NW = NC * NS
NBUF = 8
TBUF = 2
VBUF = 3
WBUF = 2


def _make_detile(v, d):
    nb = 128
    full = v // nb
    base, extra = full // NW, full % NW
    mesh = plsc.VectorSubcoreMesh(core_axis_name="c", subcore_axis_name="s")

    @functools.partial(
        pl.kernel,
        out_type=jax.ShapeDtypeStruct((full, d, nb), jnp.float32),
        mesh=mesh,
        scratch_types=[
            pltpu.VMEM((VBUF, d, nb), jnp.float32),
            pltpu.SemaphoreType.DMA((VBUF,)),
            pltpu.SemaphoreType.DMA((VBUF,)),
        ],
        compiler_params=pltpu.CompilerParams(use_tc_tiling_on_sc=True,
                                             needs_layout_passes=False),
    )
    def detile_kernel(tT_hbm, out_hbm, src_v, gsem, osem):
        wid = lax.axis_index("s") * NC + lax.axis_index("c")
        n = base + jnp.where(wid < extra, 1, 0)

        def blk_of(j):
            return j * NW + wid

        for p in range(VBUF):
            pltpu.async_copy(tT_hbm.at[:, pl.ds(blk_of(p) * nb, nb)],
                             src_v.at[p], gsem.at[p])

        @pl.loop(0, base + 1)
        def step(j):
            @pl.when(j < n)
            def _do():
                p = lax.rem(j, VBUF)
                pltpu.make_async_copy(
                    tT_hbm.at[:, pl.ds(blk_of(j) * nb, nb)],
                    src_v.at[p], gsem.at[p]).wait()
                pltpu.async_copy(src_v.at[p], out_hbm.at[blk_of(j)],
                                 osem.at[p])
                pv = lax.rem(j + VBUF - 1, VBUF)
                jn = j - 1 + VBUF

                @pl.when((j >= 1) & (jn < n))
                def _refire():
                    pltpu.make_async_copy(src_v.at[pv],
                                          out_hbm.at[blk_of(j - 1)],
                                          osem.at[pv]).wait()
                    pltpu.async_copy(
                        tT_hbm.at[:, pl.ds(blk_of(jn) * nb, nb)],
                        src_v.at[pv], gsem.at[pv])

        @pl.loop(0, VBUF)
        def _drain(k):
            j = n - VBUF + k

            @pl.when(j >= 0)
            def _d():
                pltpu.make_async_copy(src_v.at[lax.rem(j, VBUF)],
                                      out_hbm.at[blk_of(j)],
                                      osem.at[lax.rem(j, VBUF)]).wait()

    return detile_kernel


def _make_transpose(v, d):
    nb = 128
    full = v // nb
    rem = v - full * nb
    base, extra = full // NW, full % NW
    mesh = plsc.VectorSubcoreMesh(core_axis_name="c", subcore_axis_name="s")

    @functools.partial(
        pl.kernel,
        out_type=jax.ShapeDtypeStruct((v // 2, 2 * d), jnp.float32),
        mesh=mesh,
        scratch_types=[
            pltpu.VMEM((VBUF, d, nb), jnp.float32),
            pltpu.VMEM((WBUF, nb // 2, 2 * d + 1), jnp.float32),
            pltpu.SemaphoreType.DMA((VBUF,)),
            pltpu.SemaphoreType.DMA((WBUF,)),
        ],
        compiler_params=pltpu.CompilerParams(use_tc_tiling_on_sc=False,
                                             needs_layout_passes=False),
    )
    def transpose_kernel(blk_hbm, tail_hbm, out_hbm, src_v, dst_v,
                         gsem, osem):
        wid = lax.axis_index("s") * NC + lax.axis_index("c")
        n = base + jnp.where(wid < extra, 1, 0)

        io16 = lax.iota(jnp.int32, 16)
        m_ids = [(io16 + 16 * g) >> 1 for g in range(8)]
        q_off = [((io16 + 16 * g) & 1) * d for g in range(8)]

        def blk_of(j):
            return j * NW + wid

        for p in range(VBUF):
            pltpu.async_copy(blk_hbm.at[blk_of(p)], src_v.at[p], gsem.at[p])

        def transpose_block(src, dst, ngroups):
            @pl.loop(0, d, step=2)
            def _row(dd0):
                for du in range(2):
                    dd = dd0 + du
                    for g in range(ngroups):
                        vals = src[dd, pl.ds(16 * g, 16)]
                        plsc.store_scatter(dst, [m_ids[g], q_off[g] + dd],
                                           vals)

        def write_out(w, m0, nrows):
            pltpu.async_copy(dst_v.at[w, pl.ds(0, nrows), pl.ds(0, 2 * d)],
                             out_hbm.at[pl.ds(m0, nrows)],
                             osem.at[w])

        def wait_out(w, m0, nrows):
            pltpu.make_async_copy(
                dst_v.at[w, pl.ds(0, nrows), pl.ds(0, 2 * d)],
                out_hbm.at[pl.ds(m0, nrows)],
                osem.at[w]).wait()

        @pl.loop(0, base + 1)
        def step(j):
            @pl.when(j < n)
            def _do():
                p = lax.rem(j, VBUF)
                w = lax.rem(j, WBUF)
                pltpu.make_async_copy(blk_hbm.at[blk_of(j)],
                                      src_v.at[p], gsem.at[p]).wait()

                @pl.when(j >= WBUF)
                def _w():
                    wait_out(w, blk_of(j - WBUF) * (nb // 2), nb // 2)

                transpose_block(src_v.at[p], dst_v.at[w], 8)

                @pl.when(j + VBUF < n)
                def _refire():
                    pltpu.async_copy(blk_hbm.at[blk_of(j + VBUF)],
                                     src_v.at[p], gsem.at[p])

                write_out(w, blk_of(j) * (nb // 2), nb // 2)

        @pl.loop(0, WBUF)
        def _drain(k):
            j = n - WBUF + k

            @pl.when(j >= 0)
            def _d():
                wait_out(lax.rem(j, WBUF), blk_of(j) * (nb // 2), nb // 2)

        if rem:
            @pl.when(wid == NW - 1)
            def _tail():
                pltpu.sync_copy(tail_hbm, src_v.at[0])
                transpose_block(src_v.at[0], dst_v.at[0], rem // 16)
                pltpu.sync_copy(
                    dst_v.at[0, pl.ds(0, rem // 2), pl.ds(0, 2 * d)],
                    out_hbm.at[pl.ds(full * (nb // 2), rem // 2)])

    return transpose_kernel


def _make_gather(bsz, seq, d):
    tb = bsz // 128
    assert tb == NW and d % 8 == 0 and seq % NBUF == 0
    d8 = d // 8
    mesh = plsc.VectorSubcoreMesh(core_axis_name="c", subcore_axis_name="s")

    @functools.partial(
        pl.kernel,
        out_type=jax.ShapeDtypeStruct((seq, d8, tb, 8, 128), jnp.float32),
        mesh=mesh,
        scratch_types=[
            pltpu.VMEM((seq // 8, 8, 128), jnp.int32),
            pltpu.VMEM((NBUF, 128, d), jnp.float32),
            pltpu.VMEM((TBUF, d8, 8, 129), jnp.float32),
            pltpu.SemaphoreType.DMA((NBUF,)),
            pltpu.SemaphoreType.DMA((TBUF,)),
        ],
        compiler_params=pltpu.CompilerParams(use_tc_tiling_on_sc=False,
                                             needs_layout_passes=False),
    )
    def gather_kernel(idx_hbm, table_hbm, out_hbm, idx_v, rows_v, t_v,
                      gsem, osem):
        wid = lax.axis_index("s") * NC + lax.axis_index("c")
        pltpu.sync_copy(idx_hbm.at[:, wid], idx_v)

        io16 = lax.iota(jnp.int32, 16)
        i0s = [(io16 + 16 * g) >> 3 for g in range(4)]
        i1s = [(io16 + 16 * g) & 7 for g in range(4)]

        for b in range(NBUF):
            pltpu.async_copy(table_hbm.at[idx_v.at[b // 8, b % 8]],
                             rows_v.at[b], gsem.at[b])

        @pl.loop(0, seq, step=NBUF)
        def step(j):
            for b in range(NBUF):
                i = j + b
                t = b % TBUF
                pltpu.make_async_copy(table_hbm.at[idx_v.at[i // 8, i % 8]],
                                      rows_v.at[b], gsem.at[b]).wait()

                @pl.when(i >= TBUF)
                def _wait_write():
                    pltpu.make_async_copy(
                        t_v.at[t, :, :, pl.ds(0, 128)],
                        out_hbm.at[i - TBUF, pl.ds(0, d8), wid],
                        osem.at[t]).wait()

                rows = rows_v.at[b]
                tdst = t_v.at[t]

                @pl.loop(0, 128, step=4)
                def _col(c0):
                    for cc in range(4):
                        c = c0 + cc
                        cvec = jnp.full((16,), c, jnp.int32)
                        for g in range(4):
                            vals = rows[c, pl.ds(16 * g, 16)]
                            plsc.store_scatter(
                                tdst, [i0s[g], i1s[g], cvec], vals)

                @pl.when(i + NBUF < seq)
                def _refire():
                    pltpu.async_copy(
                        table_hbm.at[idx_v.at[(i + NBUF) // 8,
                                              (i + NBUF) % 8]],
                        rows_v.at[b], gsem.at[b])

                pltpu.async_copy(t_v.at[t, :, :, pl.ds(0, 128)],
                                 out_hbm.at[i, pl.ds(0, d8), wid],
                                 osem.at[t])

        for k in range(TBUF):
            i = seq - TBUF + k
            pltpu.make_async_copy(t_v.at[i % TBUF, :, :, pl.ds(0, 128)],
                                  out_hbm.at[i, pl.ds(0, d8), wid],
                                  osem.at[i % TBUF]).wait()

    return gather_kernel


def kernel(inputs, table):
    bsz, seq = inputs.shape
    v, d = table.shape
    rem = v % 128
    tail = (jnp.pad(table[v - rem:].T, ((0, 0), (0, 128 - rem)))
            if rem else jnp.zeros((d, 128), table.dtype))
    blocks = _make_detile(v, d)(table.T)
    tbl_lin = _make_transpose(v, d)(blocks, tail).reshape(v, d)
    idx4 = (inputs.astype(jnp.int32)
            .reshape(bsz // 128, 128, seq // 8, 8)
            .transpose(2, 0, 3, 1))
    out5 = _make_gather(bsz, seq, d)(idx4, tbl_lin)
    out = out5.transpose(2, 4, 0, 1, 3).reshape(bsz, seq, d)
    return out


# speedup vs baseline: 1.6322x; 1.1881x over previous
import functools

import jax
import jax.numpy as jnp
from jax import lax
from jax.experimental import pallas as pl
from jax.experimental.pallas import tpu as pltpu
from jax.experimental.pallas import tpu_sc as plsc

NC, NS = 2, 16
NW = NC * NS
NBUF = 8
TBUF = 2


def _make_gather(bsz, seq, d):
    tb = bsz // 128
    assert tb == NW and d % 8 == 0 and seq % NBUF == 0
    d8 = d // 8
    mesh = plsc.VectorSubcoreMesh(core_axis_name="c", subcore_axis_name="s")

    @functools.partial(
        pl.kernel,
        out_type=jax.ShapeDtypeStruct((seq, d8, tb, 8, 128), jnp.float32),
        mesh=mesh,
        scratch_types=[
            pltpu.VMEM((seq // 8, 8, 128), jnp.int32),
            pltpu.VMEM((NBUF, 128, d), jnp.float32),
            pltpu.VMEM((TBUF, d8, 8, 129), jnp.float32),
            pltpu.SemaphoreType.DMA((NBUF,)),
            pltpu.SemaphoreType.DMA((TBUF,)),
        ],
        compiler_params=pltpu.CompilerParams(use_tc_tiling_on_sc=False,
                                             needs_layout_passes=False),
    )
    def gather_kernel(idx_hbm, table_hbm, out_hbm, idx_v, rows_v, t_v,
                      gsem, osem):
        wid = lax.axis_index("s") * NC + lax.axis_index("c")
        pltpu.sync_copy(idx_hbm.at[:, wid], idx_v)

        io16 = lax.iota(jnp.int32, 16)
        i0s = [(io16 + 16 * g) >> 3 for g in range(4)]
        i1s = [(io16 + 16 * g) & 7 for g in range(4)]

        for b in range(NBUF):
            pltpu.async_copy(table_hbm.at[idx_v.at[b // 8, b % 8]],
                             rows_v.at[b], gsem.at[b])

        @pl.loop(0, seq, step=NBUF)
        def step(j):
            for b in range(NBUF):
                i = j + b
                t = b % TBUF
                pltpu.make_async_copy(table_hbm.at[idx_v.at[i // 8, i % 8]],
                                      rows_v.at[b], gsem.at[b]).wait()

                @pl.when(i >= TBUF)
                def _wait_write():
                    pltpu.make_async_copy(
                        t_v.at[t, :, :, pl.ds(0, 128)],
                        out_hbm.at[i - TBUF, pl.ds(0, d8), wid],
                        osem.at[t]).wait()

                rows = rows_v.at[b]
                tdst = t_v.at[t]

                @pl.loop(0, 128, step=4)
                def _col(c0):
                    for cc in range(4):
                        c = c0 + cc
                        cvec = jnp.full((16,), c, jnp.int32)
                        for g in range(4):
                            vals = rows[c, pl.ds(16 * g, 16)]
                            plsc.store_scatter(
                                tdst, [i0s[g], i1s[g], cvec], vals)

                @pl.when(i + NBUF < seq)
                def _refire():
                    pltpu.async_copy(
                        table_hbm.at[idx_v.at[(i + NBUF) // 8,
                                              (i + NBUF) % 8]],
                        rows_v.at[b], gsem.at[b])

                pltpu.async_copy(t_v.at[t, :, :, pl.ds(0, 128)],
                                 out_hbm.at[i, pl.ds(0, d8), wid],
                                 osem.at[t])

        for k in range(TBUF):
            i = seq - TBUF + k
            pltpu.make_async_copy(t_v.at[i % TBUF, :, :, pl.ds(0, 128)],
                                  out_hbm.at[i, pl.ds(0, d8), wid],
                                  osem.at[i % TBUF]).wait()

    return gather_kernel


def kernel(inputs, table):
    bsz, seq = inputs.shape
    _, d = table.shape
    idx4 = (inputs.astype(jnp.int32)
            .reshape(bsz // 128, 128, seq // 8, 8)
            .transpose(2, 0, 3, 1))
    out5 = _make_gather(bsz, seq, d)(idx4, table)
    out = out5.transpose(2, 4, 0, 1, 3).reshape(bsz, seq, d)
    return out
